# Initial kernel scaffold; baseline (speedup 1.0000x reference)
import jax
import jax.numpy as jnp
from jax.experimental import pallas as pl


def kernel(edge_index, edge_weight, emb):
    raise NotImplementedError("write your pallas kernel here")


# speedup vs baseline: 1.2689x; 1.2689x over previous
import functools

import jax
import jax.numpy as jnp
from jax.experimental import pallas as pl
from jax.experimental.pallas import tpu as pltpu

_EPS = 1e-8
_LANE = 128
_BAND = 512
_JSPLIT = 2
_JBLK = 2048
_VMEM_LIMIT = 52 * 1024 * 1024


def _ceil_to(x, m):
    return (x + m - 1) // m * m


def _sym_deg_mean_kernel(row_ref, col_ref, emb_ref, sym_ref, d_ref, mean_ref,
                         *, inv_n):
    i = pl.program_id(0)
    j = pl.program_id(1)
    sym = jnp.maximum(row_ref[...], col_ref[...].T)
    sym_ref[...] = sym.astype(sym_ref.dtype)
    part = jnp.sum(sym, axis=-1, keepdims=True)

    @pl.when(j == 0)
    def _():
        d_ref[...] = jnp.zeros_like(d_ref)

    d_ref[...] += part

    @pl.when(jnp.logical_and(i == 0, j == 0))
    def _():
        mean_ref[...] = jnp.zeros_like(mean_ref)

    @pl.when(j == 0)
    def _():
        m = jnp.sum(emb_ref[...].astype(jnp.float32), axis=-2, keepdims=True)
        mean_ref[...] += (m * inv_n).astype(mean_ref.dtype)


def _scale_kernel(sym_ref, dr_ref, dc_ref, o_ref):
    sym = sym_ref[...].astype(jnp.float32)
    o_ref[...] = (dr_ref[...] * sym) * dc_ref[...]


def kernel(edge_index, edge_weight, emb):
    n, h = emb.shape
    out_dtype = emb.dtype

    adj = jnp.zeros((n, n), jnp.float32).at[edge_index[0], edge_index[1]].add(
        edge_weight.astype(jnp.float32))

    band = min(_BAND, _ceil_to(n, _LANE))
    n_pad = _ceil_to(n, band)
    h_pad = _ceil_to(h, _LANE)
    if n_pad != n:
        adj = jnp.pad(adj, ((0, n_pad - n), (0, n_pad - n)))
    emb_p = emb
    if n_pad != n or h_pad != h:
        emb_p = jnp.pad(emb, ((0, n_pad - n), (0, h_pad - h)))
    gi = n_pad // band
    jb = n_pad // _JSPLIT if n_pad % _JSPLIT == 0 else n_pad
    gj = n_pad // jb

    sym16, d, mean = pl.pallas_call(
        functools.partial(_sym_deg_mean_kernel, inv_n=1.0 / n),
        out_shape=(jax.ShapeDtypeStruct((n_pad, n_pad), jnp.bfloat16),
                   jax.ShapeDtypeStruct((n_pad, 1), jnp.float32),
                   jax.ShapeDtypeStruct((1, h_pad), out_dtype)),
        grid=(gi, gj),
        in_specs=[pl.BlockSpec((band, jb), lambda i, j: (i, j)),
                  pl.BlockSpec((jb, band), lambda i, j: (j, i)),
                  pl.BlockSpec((band, h_pad), lambda i, j: (i, 0))],
        out_specs=(pl.BlockSpec((band, jb), lambda i, j: (i, j)),
                   pl.BlockSpec((band, 1), lambda i, j: (i, 0)),
                   pl.BlockSpec((1, h_pad), lambda i, j: (0, 0))),
        compiler_params=pltpu.CompilerParams(
            dimension_semantics=("arbitrary", "arbitrary"),
            vmem_limit_bytes=_VMEM_LIMIT),
        cost_estimate=pl.CostEstimate(
            flops=int(3 * n_pad * n_pad + 2 * n_pad * h_pad),
            transcendentals=0,
            bytes_accessed=int(10 * n_pad * n_pad + 4 * n_pad * h_pad)),
    )(adj, adj, emb_p)

    dinv = 1.0 / (jnp.sqrt(d) + _EPS)
    dinv_col = dinv.reshape(1, n_pad)

    jblk = min(_JBLK, n_pad)
    gj2 = n_pad // jblk
    adj_norm = pl.pallas_call(
        _scale_kernel,
        out_shape=jax.ShapeDtypeStruct((n_pad, n_pad), out_dtype),
        grid=(gi, gj2),
        in_specs=[pl.BlockSpec((band, jblk), lambda i, j: (i, j)),
                  pl.BlockSpec((band, 1), lambda i, j: (i, 0)),
                  pl.BlockSpec((1, jblk), lambda i, j: (0, j))],
        out_specs=pl.BlockSpec((band, jblk), lambda i, j: (i, j)),
        compiler_params=pltpu.CompilerParams(
            dimension_semantics=("parallel", "parallel"),
            vmem_limit_bytes=_VMEM_LIMIT),
        cost_estimate=pl.CostEstimate(
            flops=int(2 * n_pad * n_pad),
            transcendentals=0,
            bytes_accessed=int(6 * n_pad * n_pad)),
    )(sym16, dinv, dinv_col)

    if n_pad != n:
        adj_norm = adj_norm[:n, :n]
    if h_pad != h:
        mean = mean[:, :h]
    return adj_norm, mean
